# baseline (speedup 1.0000x reference)
import functools

import jax
import jax.numpy as jnp
from jax import lax
from jax.experimental import pallas as pl
from jax.experimental.pallas import tpu as pltpu
from jax.experimental.pallas import tpu_sc as plsc

B = 16384
D = 64
NC = 2
NS = 16
NW = NC * NS
BPW = B // NW
CH = 128
NCH = BPW // CH
VL = 16

_mesh = plsc.VectorSubcoreMesh(core_axis_name="c", subcore_axis_name="s")


@functools.partial(
    pl.kernel,
    mesh=_mesh,
    out_type=(
        jax.ShapeDtypeStruct((B, D), jnp.float32),
        jax.ShapeDtypeStruct((B, D), jnp.float32),
        jax.ShapeDtypeStruct((B, D), jnp.float32),
    ),
    scratch_types=(
        pltpu.VMEM((4, 128), jnp.int32),
        pltpu.VMEM((4, 128), jnp.int32),
        pltpu.VMEM((4, 128), jnp.int32),
        pltpu.SemaphoreType.DMA,
        pltpu.SemaphoreType.DMA,
        pltpu.SemaphoreType.DMA,
    ),
)
def _gather3(uid, iid, cid, ut, it, ct, ou, oi, oc,
             l0, l1, l2, s0, s1, s2):
    wid = lax.axis_index("s") * NC + lax.axis_index("c")
    base = wid * BPW
    srcs = (uid, iid, cid)
    tabs = (ut, it, ct)
    outs = (ou, oi, oc)
    lands = (l0, l1, l2)
    sems = (s0, s1, s2)

    for t in range(3):
        pltpu.sync_copy(srcs[t].at[pl.ds(wid * 4, 4)], lands[t])

    for t in range(3):
        def body(c, carry, t=t):
            for g in range(CH // VL):
                mv = lands[t][c, pl.ds(g * VL, VL)]
                for u in range(VL):
                    r = c * CH + g * VL + u
                    pltpu.async_copy(
                        tabs[t].at[pl.ds(mv[u], 1)],
                        outs[t].at[pl.ds(base + r, 1)],
                        sems[t])
            return carry

        lax.fori_loop(0, NCH, body, 0)

    for t in range(3):
        pltpu.make_async_copy(
            tabs[t].at[pl.ds(0, BPW)],
            outs[t].at[pl.ds(base, BPW)],
            sems[t]).wait()


def kernel(user_id, item_id, category_id, user_table, item_table, cat_table):
    uid = user_id.reshape(NW * 4, 128)
    iid = item_id.reshape(NW * 4, 128)
    cid = category_id.reshape(NW * 4, 128)
    return _gather3(uid, iid, cid, user_table, item_table, cat_table)


# speedup vs baseline: 1.9415x; 1.9415x over previous
import functools

import jax
import jax.numpy as jnp
from jax import lax
from jax.experimental import pallas as pl
from jax.experimental.pallas import tpu as pltpu
from jax.experimental.pallas import tpu_sc as plsc

B = 16384
D = 64
NC = 2
NS = 16
NW = NC * NS
BPW = B // NW
CH = 64
NCH = BPW // CH
VL = 16
NQ = 4

_mesh = plsc.VectorSubcoreMesh(core_axis_name="c", subcore_axis_name="s")


@functools.partial(
    pl.kernel,
    mesh=_mesh,
    out_type=(
        jax.ShapeDtypeStruct((B, D), jnp.float32),
        jax.ShapeDtypeStruct((B, D), jnp.float32),
        jax.ShapeDtypeStruct((B, D), jnp.float32),
    ),
    scratch_types=(
        pltpu.VMEM((4, 128), jnp.int32),
        pltpu.VMEM((4, 128), jnp.int32),
        pltpu.VMEM((4, 128), jnp.int32),
        pltpu.VMEM((2, CH, D), jnp.float32),
    ) + (pltpu.SemaphoreType.DMA,) * (2 * NQ)
      + (
        pltpu.SemaphoreType.DMA,
        pltpu.SemaphoreType.DMA,
    ),
)
def _gather3(uid, iid, cid, ut, it, ct, ou, oi, oc,
             l0, l1, l2, buf, *sems):
    wid = lax.axis_index("s") * NC + lax.axis_index("c")
    base = wid * BPW
    srcs = (uid, iid, cid)
    tabs = (ut, it, ct)
    outs = (ou, oi, oc)
    lands = (l0, l1, l2)
    gsems = (sems[:NQ], sems[NQ:2 * NQ])
    wsems = sems[2 * NQ:]

    for t in range(3):
        pltpu.sync_copy(srcs[t].at[pl.ds(wid * 4, 4)], lands[t])

    def fire(t, c, slot):
        hs = []
        for g in range(CH // VL):
            mv = lands[t][(c * CH + g * VL) // 128,
                          pl.ds((c * CH + g * VL) % 128, VL)]
            for u in range(VL):
                r = g * VL + u
                hs.append(pltpu.async_copy(
                    tabs[t].at[pl.ds(mv[u], 1)],
                    buf.at[slot].at[pl.ds(r, 1)],
                    gsems[slot][r % NQ]))
        return hs

    for t in range(3):
        def body(i, carry, t=t):
            gh = [None, None]
            wh = [None, None]
            for b in range(2):
                gh[b] = fire(t, 2 * i + b, b)
            for b in range(2):
                c = 2 * i + b
                for h in gh[b]:
                    h.wait()
                wh[b] = pltpu.async_copy(
                    buf.at[b], outs[t].at[pl.ds(base + c * CH, CH)],
                    wsems[b])
            wh[0].wait()
            wh[1].wait()
            return carry

        lax.fori_loop(0, NCH // 2, body, 0)


def kernel(user_id, item_id, category_id, user_table, item_table, cat_table):
    uid = user_id.reshape(NW * 4, 128)
    iid = item_id.reshape(NW * 4, 128)
    cid = category_id.reshape(NW * 4, 128)
    return _gather3(uid, iid, cid, user_table, item_table, cat_table)
